# baseline (speedup 1.0000x reference)
import functools

import jax
import jax.numpy as jnp
from jax import lax
from jax.experimental import pallas as pl
from jax.experimental.pallas import tpu as pltpu
from jax.experimental.pallas import tpu_sc as plsc

N = 10000
E = 320000
D = 128
H = 128

NC = 2
NS = 16
NW = NC * NS

CHUNK = 125
CPT = 80
HALF = CPT // 2
EPT = CPT * CHUNK

N_PAD = 10112

RPT = N_PAD // NS

_sc_mesh = plsc.VectorSubcoreMesh(core_axis_name="c", subcore_axis_name="s")


def _sc_agg_body(feat_hbm, src_hbm, dst_hbm, zero_hbm, out_hbm,
                 src_v, dst_v, rows0, rows1, agg_sh, sem0, sem1):
    c = lax.axis_index("c")
    s = lax.axis_index("s")
    wid = c * NS + s

    pltpu.sync_copy(zero_hbm.at[pl.ds(s * RPT, RPT)],
                    agg_sh.at[pl.ds(s * RPT, RPT)])
    plsc.subcore_barrier()

    def gather(j, buf, sem):
        return pltpu.async_copy(feat_hbm.at[src_v.at[j]], buf, sem)

    def gather_wait(j, buf, sem):
        pltpu.make_async_copy(feat_hbm.at[src_v.at[j]], buf, sem).wait()

    def scatter(j, buf):
        pltpu.sync_copy(buf, agg_sh.at[dst_v.at[j]], add=True)

    for h in range(CPT // HALF):
        pltpu.sync_copy(src_hbm.at[wid, pl.ds(h * HALF, HALF)], src_v)
        pltpu.sync_copy(dst_hbm.at[wid, pl.ds(h * HALF, HALF)], dst_v)

        gather(0, rows0, sem0)
        gather(1, rows1, sem1)

        def step(i, carry):
            g = 2 * i
            gather_wait(g, rows0, sem0)
            scatter(g, rows0)
            gather(g + 2, rows0, sem0)
            gather_wait(g + 1, rows1, sem1)
            scatter(g + 1, rows1)
            gather(g + 3, rows1, sem1)
            return carry

        lax.fori_loop(0, HALF // 2 - 1, step, 0)

        gather_wait(HALF - 2, rows0, sem0)
        scatter(HALF - 2, rows0)
        gather_wait(HALF - 1, rows1, sem1)
        scatter(HALF - 1, rows1)

    plsc.subcore_barrier()
    pltpu.sync_copy(agg_sh.at[pl.ds(s * RPT, RPT)],
                    out_hbm.at[c, pl.ds(s * RPT, RPT)])


_sc_agg = pl.kernel(
    _sc_agg_body,
    out_type=jax.ShapeDtypeStruct((NC, N_PAD, D), jnp.float32),
    mesh=_sc_mesh,
    scratch_types=[
        pltpu.VMEM((HALF, CHUNK), jnp.int32),
        pltpu.VMEM((HALF, CHUNK), jnp.int32),
        pltpu.VMEM((CHUNK, D), jnp.float32),
        pltpu.VMEM((CHUNK, D), jnp.float32),
        pltpu.VMEM_SHARED((N_PAD, D), jnp.float32),
        pltpu.SemaphoreType.DMA,
        pltpu.SemaphoreType.DMA,
    ],
)


def _tc_mlp_body(feat_ref, agg_ref, w1_ref, b1_ref, w2_ref, b2_ref,
                 gamma_ref, beta_ref, out_ref):
    agg = agg_ref[0, :N, :] + agg_ref[1, :N, :]
    h = 2.0 * feat_ref[...] + agg
    h = jnp.dot(h, w1_ref[...], preferred_element_type=jnp.float32) + b1_ref[...]
    h = jnp.maximum(h, 0.0)
    h = jnp.dot(h, w2_ref[...], preferred_element_type=jnp.float32) + b2_ref[...]
    mean = jnp.mean(h, axis=0, keepdims=True)
    var = jnp.mean((h - mean) * (h - mean), axis=0, keepdims=True)
    h = (h - mean) * lax.rsqrt(var + 1e-5) * gamma_ref[...] + beta_ref[...]
    out_ref[...] = jnp.maximum(h, 0.0)


_tc_mlp = pl.pallas_call(
    _tc_mlp_body,
    out_shape=jax.ShapeDtypeStruct((N, H), jnp.float32),
)


def kernel(features, edge_index, W1, b1, W2, b2, gamma, beta):
    src_p = edge_index[0].reshape(NW, CPT, CHUNK)
    dst_p = edge_index[1].reshape(NW, CPT, CHUNK)
    zeros = jnp.zeros((N_PAD, D), jnp.float32)
    agg = _sc_agg(features, src_p, dst_p, zeros)
    return _tc_mlp(features, agg,
                   W1, b1.reshape(1, H), W2, b2.reshape(1, H),
                   gamma.reshape(1, H), beta.reshape(1, H))


# speedup vs baseline: 1.1080x; 1.1080x over previous
import functools

import jax
import jax.numpy as jnp
from jax import lax
from jax.experimental import pallas as pl
from jax.experimental.pallas import tpu as pltpu
from jax.experimental.pallas import tpu_sc as plsc

N = 10000
E = 320000
D = 128
H = 128

NC = 2
NS = 16
NW = NC * NS

CHUNK = 125
CPT = 80
HALF = CPT // 2
EPT = CPT * CHUNK

N_PAD = 10112

RPT = N_PAD // NS

_sc_mesh = plsc.VectorSubcoreMesh(core_axis_name="c", subcore_axis_name="s")


ZCOPY = 120


def _sc_agg_body(edge_hbm, feat_hbm, out_hbm,
                 src_v, dst_v, rows0, rows1, agg_sh, sem0, sem1):
    c = lax.axis_index("c")
    s = lax.axis_index("s")
    wid = c * NS + s

    zvec = jnp.zeros((16,), jnp.float32)

    def zrow(r, carry):
        def zcol(k, carry2):
            rows0[r, pl.ds(k * 16, 16)] = zvec
            return carry2
        return lax.fori_loop(0, D // 16, zcol, carry)

    lax.fori_loop(0, ZCOPY, zrow, 0)
    base = s * RPT
    for z in range(RPT // ZCOPY):
        pltpu.sync_copy(rows0.at[pl.ds(0, ZCOPY)],
                        agg_sh.at[pl.ds(base + z * ZCOPY, ZCOPY)])
    rem = RPT % ZCOPY
    pltpu.sync_copy(rows0.at[pl.ds(0, rem)],
                    agg_sh.at[pl.ds(base + RPT - rem, rem)])
    plsc.subcore_barrier()

    def gather(j, buf, sem):
        return pltpu.async_copy(feat_hbm.at[src_v.at[j]], buf, sem)

    def gather_wait(j, buf, sem):
        pltpu.make_async_copy(feat_hbm.at[src_v.at[j]], buf, sem).wait()

    def scatter(j, buf):
        pltpu.sync_copy(buf, agg_sh.at[dst_v.at[j]], add=True)

    for h in range(CPT // HALF):
        pltpu.sync_copy(edge_hbm.at[0, wid, pl.ds(h * HALF, HALF)], src_v)
        pltpu.sync_copy(edge_hbm.at[1, wid, pl.ds(h * HALF, HALF)], dst_v)

        gather(0, rows0, sem0)
        gather(1, rows1, sem1)

        def step(i, carry):
            g = 2 * i
            gather_wait(g, rows0, sem0)
            scatter(g, rows0)
            gather(g + 2, rows0, sem0)
            gather_wait(g + 1, rows1, sem1)
            scatter(g + 1, rows1)
            gather(g + 3, rows1, sem1)
            return carry

        lax.fori_loop(0, HALF // 2 - 1, step, 0)

        gather_wait(HALF - 2, rows0, sem0)
        scatter(HALF - 2, rows0)
        gather_wait(HALF - 1, rows1, sem1)
        scatter(HALF - 1, rows1)

    plsc.subcore_barrier()
    pltpu.sync_copy(agg_sh.at[pl.ds(s * RPT, RPT)],
                    out_hbm.at[c, pl.ds(s * RPT, RPT)])


_sc_agg = pl.kernel(
    _sc_agg_body,
    out_type=jax.ShapeDtypeStruct((NC, N_PAD, D), jnp.float32),
    mesh=_sc_mesh,
    scratch_types=[
        pltpu.VMEM((HALF, CHUNK), jnp.int32),
        pltpu.VMEM((HALF, CHUNK), jnp.int32),
        pltpu.VMEM((CHUNK, D), jnp.float32),
        pltpu.VMEM((CHUNK, D), jnp.float32),
        pltpu.VMEM_SHARED((N_PAD, D), jnp.float32),
        pltpu.SemaphoreType.DMA,
        pltpu.SemaphoreType.DMA,
    ],
)


def _tc_mlp_body(feat_ref, agg_ref, w1_ref, b1_ref, w2_ref, b2_ref,
                 gamma_ref, beta_ref, out_ref):
    agg = agg_ref[0, :N, :] + agg_ref[1, :N, :]
    h = 2.0 * feat_ref[...] + agg
    h = jnp.dot(h, w1_ref[...], preferred_element_type=jnp.float32) + b1_ref[...]
    h = jnp.maximum(h, 0.0)
    h = jnp.dot(h, w2_ref[...], preferred_element_type=jnp.float32) + b2_ref[...]
    mean = jnp.mean(h, axis=0, keepdims=True)
    var = jnp.mean((h - mean) * (h - mean), axis=0, keepdims=True)
    h = (h - mean) * lax.rsqrt(var + 1e-5) * gamma_ref[...] + beta_ref[...]
    out_ref[...] = jnp.maximum(h, 0.0)


_tc_mlp = pl.pallas_call(
    _tc_mlp_body,
    out_shape=jax.ShapeDtypeStruct((N, H), jnp.float32),
)


def kernel(features, edge_index, W1, b1, W2, b2, gamma, beta):
    edges = edge_index.reshape(2, NW, CPT, CHUNK)
    agg = _sc_agg(edges, features)
    return _tc_mlp(features, agg,
                   W1, b1.reshape(1, H), W2, b2.reshape(1, H),
                   gamma.reshape(1, H), beta.reshape(1, H))


# speedup vs baseline: 1.1110x; 1.0028x over previous
import functools

import jax
import jax.numpy as jnp
from jax import lax
from jax.experimental import pallas as pl
from jax.experimental.pallas import tpu as pltpu
from jax.experimental.pallas import tpu_sc as plsc

N = 10000
E = 320000
D = 128
H = 128

NC = 2
NS = 16
NW = NC * NS

CHUNK = 125
CPT = 80
HALF = CPT // 2
EPT = CPT * CHUNK

N_PAD = 10112

RPT = N_PAD // NS

_sc_mesh = plsc.VectorSubcoreMesh(core_axis_name="c", subcore_axis_name="s")


ZCOPY = 120


def _sc_agg_body(edge_hbm, feat_hbm, out_hbm,
                 src_v, dst_v, rows0, rows1, agg_sh, sem0, sem1):
    c = lax.axis_index("c")
    s = lax.axis_index("s")
    wid = c * NS + s

    zvec = jnp.zeros((16,), jnp.float32)

    def zrow(r, carry):
        def zcol(k, carry2):
            rows0[r, pl.ds(k * 16, 16)] = zvec
            return carry2
        return lax.fori_loop(0, D // 16, zcol, carry)

    lax.fori_loop(0, ZCOPY, zrow, 0)
    base = s * RPT
    for z in range(RPT // ZCOPY):
        pltpu.sync_copy(rows0.at[pl.ds(0, ZCOPY)],
                        agg_sh.at[pl.ds(base + z * ZCOPY, ZCOPY)])
    rem = RPT % ZCOPY
    pltpu.sync_copy(rows0.at[pl.ds(0, rem)],
                    agg_sh.at[pl.ds(base + RPT - rem, rem)])
    plsc.subcore_barrier()

    SPL = 64

    def gather(j, buf, sem):
        pltpu.async_copy(feat_hbm.at[src_v.at[j, pl.ds(0, SPL)]],
                         buf.at[pl.ds(0, SPL)], sem)
        pltpu.async_copy(feat_hbm.at[src_v.at[j, pl.ds(SPL, CHUNK - SPL)]],
                         buf.at[pl.ds(SPL, CHUNK - SPL)], sem)

    def gather_wait(j, buf, sem):
        pltpu.make_async_copy(feat_hbm.at[src_v.at[j, pl.ds(0, SPL)]],
                              buf.at[pl.ds(0, SPL)], sem).wait()
        pltpu.make_async_copy(feat_hbm.at[src_v.at[j, pl.ds(SPL, CHUNK - SPL)]],
                              buf.at[pl.ds(SPL, CHUNK - SPL)], sem).wait()

    def scatter(j, buf):
        pltpu.sync_copy(buf, agg_sh.at[dst_v.at[j]], add=True)

    for h in range(CPT // HALF):
        pltpu.sync_copy(edge_hbm.at[0, wid, pl.ds(h * HALF, HALF)], src_v)
        pltpu.sync_copy(edge_hbm.at[1, wid, pl.ds(h * HALF, HALF)], dst_v)

        gather(0, rows0, sem0)
        gather(1, rows1, sem1)

        def step(i, carry):
            g = 2 * i
            gather_wait(g, rows0, sem0)
            scatter(g, rows0)
            gather(g + 2, rows0, sem0)
            gather_wait(g + 1, rows1, sem1)
            scatter(g + 1, rows1)
            gather(g + 3, rows1, sem1)
            return carry

        lax.fori_loop(0, HALF // 2 - 1, step, 0)

        gather_wait(HALF - 2, rows0, sem0)
        scatter(HALF - 2, rows0)
        gather_wait(HALF - 1, rows1, sem1)
        scatter(HALF - 1, rows1)

    plsc.subcore_barrier()
    pltpu.sync_copy(agg_sh.at[pl.ds(s * RPT, RPT)],
                    out_hbm.at[c, pl.ds(s * RPT, RPT)])


_sc_agg = pl.kernel(
    _sc_agg_body,
    out_type=jax.ShapeDtypeStruct((NC, N_PAD, D), jnp.float32),
    mesh=_sc_mesh,
    scratch_types=[
        pltpu.VMEM((HALF, CHUNK), jnp.int32),
        pltpu.VMEM((HALF, CHUNK), jnp.int32),
        pltpu.VMEM((CHUNK, D), jnp.float32),
        pltpu.VMEM((CHUNK, D), jnp.float32),
        pltpu.VMEM_SHARED((N_PAD, D), jnp.float32),
        pltpu.SemaphoreType.DMA,
        pltpu.SemaphoreType.DMA,
    ],
)


def _tc_mlp_body(feat_ref, agg_ref, w1_ref, b1_ref, w2_ref, b2_ref,
                 gamma_ref, beta_ref, out_ref):
    agg = agg_ref[0, :N, :] + agg_ref[1, :N, :]
    h = 2.0 * feat_ref[...] + agg
    h = jnp.dot(h, w1_ref[...], preferred_element_type=jnp.float32) + b1_ref[...]
    h = jnp.maximum(h, 0.0)
    h = jnp.dot(h, w2_ref[...], preferred_element_type=jnp.float32) + b2_ref[...]
    mean = jnp.mean(h, axis=0, keepdims=True)
    var = jnp.mean((h - mean) * (h - mean), axis=0, keepdims=True)
    h = (h - mean) * lax.rsqrt(var + 1e-5) * gamma_ref[...] + beta_ref[...]
    out_ref[...] = jnp.maximum(h, 0.0)


_tc_mlp = pl.pallas_call(
    _tc_mlp_body,
    out_shape=jax.ShapeDtypeStruct((N, H), jnp.float32),
)


def kernel(features, edge_index, W1, b1, W2, b2, gamma, beta):
    edges = edge_index.reshape(2, NW, CPT, CHUNK)
    agg = _sc_agg(edges, features)
    return _tc_mlp(features, agg,
                   W1, b1.reshape(1, H), W2, b2.reshape(1, H),
                   gamma.reshape(1, H), beta.reshape(1, H))


# speedup vs baseline: 1.1316x; 1.0185x over previous
import functools

import numpy as np

import jax
import jax.numpy as jnp
from jax import lax
from jax.experimental import pallas as pl
from jax.experimental.pallas import tpu as pltpu
from jax.experimental.pallas import tpu_sc as plsc

N = 10000
E = 320000
D = 128
H = 128

NC = 2
NS = 16
NW = NC * NS

CHUNK = 128
CPT = 80
HALF = CPT // 2
EPT = CPT * CHUNK
E_PAD = EPT * NW

N_PAD = 10112

RPT = N_PAD // NS

_sc_mesh = plsc.VectorSubcoreMesh(core_axis_name="c", subcore_axis_name="s")


ZCOPY = 120


def _sc_agg_body(edge_hbm, feat_hbm, out_hbm,
                 src_v, dst_v, rows0, rows1, agg_sh, sem0, sem1):
    c = lax.axis_index("c")
    s = lax.axis_index("s")
    wid = c * NS + s

    zvec = jnp.zeros((16,), jnp.float32)

    def zrow(r, carry):
        def zcol(k, carry2):
            rows0[r, pl.ds(k * 16, 16)] = zvec
            return carry2
        return lax.fori_loop(0, D // 16, zcol, carry)

    lax.fori_loop(0, ZCOPY, zrow, 0)
    base = s * RPT
    for z in range(RPT // ZCOPY):
        pltpu.sync_copy(rows0.at[pl.ds(0, ZCOPY)],
                        agg_sh.at[pl.ds(base + z * ZCOPY, ZCOPY)])
    rem = RPT % ZCOPY
    pltpu.sync_copy(rows0.at[pl.ds(0, rem)],
                    agg_sh.at[pl.ds(base + RPT - rem, rem)])
    plsc.subcore_barrier()

    SPL = 64

    def gather(j, buf, sem):
        pltpu.async_copy(feat_hbm.at[src_v.at[j, pl.ds(0, SPL)]],
                         buf.at[pl.ds(0, SPL)], sem)
        pltpu.async_copy(feat_hbm.at[src_v.at[j, pl.ds(SPL, CHUNK - SPL)]],
                         buf.at[pl.ds(SPL, CHUNK - SPL)], sem)

    def gather_wait(j, buf, sem):
        pltpu.make_async_copy(feat_hbm.at[src_v.at[j, pl.ds(0, SPL)]],
                              buf.at[pl.ds(0, SPL)], sem).wait()
        pltpu.make_async_copy(feat_hbm.at[src_v.at[j, pl.ds(SPL, CHUNK - SPL)]],
                              buf.at[pl.ds(SPL, CHUNK - SPL)], sem).wait()

    def scatter(j, buf):
        pltpu.sync_copy(buf, agg_sh.at[dst_v.at[j]], add=True)

    for h in range(CPT // HALF):
        pltpu.sync_copy(edge_hbm.at[0, wid, pl.ds(h * HALF, HALF)], src_v)
        pltpu.sync_copy(edge_hbm.at[1, wid, pl.ds(h * HALF, HALF)], dst_v)

        gather(0, rows0, sem0)
        gather(1, rows1, sem1)

        def step(i, carry):
            g = 2 * i
            gather_wait(g, rows0, sem0)
            scatter(g, rows0)
            gather(g + 2, rows0, sem0)
            gather_wait(g + 1, rows1, sem1)
            scatter(g + 1, rows1)
            gather(g + 3, rows1, sem1)
            return carry

        lax.fori_loop(0, HALF // 2 - 1, step, 0)

        gather_wait(HALF - 2, rows0, sem0)
        scatter(HALF - 2, rows0)
        gather_wait(HALF - 1, rows1, sem1)
        scatter(HALF - 1, rows1)

    plsc.subcore_barrier()
    pltpu.sync_copy(agg_sh.at[pl.ds(s * RPT, RPT)],
                    out_hbm.at[c, pl.ds(s * RPT, RPT)])


_sc_agg = pl.kernel(
    _sc_agg_body,
    out_type=jax.ShapeDtypeStruct((NC, N_PAD, D), jnp.float32),
    mesh=_sc_mesh,
    scratch_types=[
        pltpu.VMEM((HALF, CHUNK), jnp.int32),
        pltpu.VMEM((HALF, CHUNK), jnp.int32),
        pltpu.VMEM((CHUNK, D), jnp.float32),
        pltpu.VMEM((CHUNK, D), jnp.float32),
        pltpu.VMEM_SHARED((N_PAD, D), jnp.float32),
        pltpu.SemaphoreType.DMA,
        pltpu.SemaphoreType.DMA,
    ],
)


def _tc_mlp_body(feat_ref, agg_ref, w1_ref, b1_ref, w2_ref, b2_ref,
                 gamma_ref, beta_ref, out_ref):
    agg = agg_ref[0, :N, :] + agg_ref[1, :N, :]
    h = 2.0 * feat_ref[...] + agg
    h = jnp.dot(h, w1_ref[...], preferred_element_type=jnp.float32) + b1_ref[...]
    h = jnp.maximum(h, 0.0)
    h = jnp.dot(h, w2_ref[...], preferred_element_type=jnp.float32) + b2_ref[...]
    mean = jnp.mean(h, axis=0, keepdims=True)
    var = jnp.mean((h - mean) * (h - mean), axis=0, keepdims=True)
    h = (h - mean) * lax.rsqrt(var + 1e-5) * gamma_ref[...] + beta_ref[...]
    out_ref[...] = jnp.maximum(h, 0.0)


_tc_mlp = pl.pallas_call(
    _tc_mlp_body,
    out_shape=jax.ShapeDtypeStruct((N, H), jnp.float32),
)


_pad_n = E_PAD - E
_pad_i = np.arange(_pad_n, dtype=np.int32)
_PAD_EDGES = np.stack([_pad_i % N, N + _pad_i % (N_PAD - N)]).reshape(2, -1, CHUNK)


def kernel(features, edge_index, W1, b1, W2, b2, gamma, beta):
    edges = jnp.concatenate(
        [edge_index.reshape(2, E // CHUNK, CHUNK), jnp.asarray(_PAD_EDGES)],
        axis=1).reshape(2, NW, CPT, CHUNK)
    agg = _sc_agg(edges, features)
    return _tc_mlp(features, agg,
                   W1, b1.reshape(1, H), W2, b2.reshape(1, H),
                   gamma.reshape(1, H), beta.reshape(1, H))


# speedup vs baseline: 1.1423x; 1.0094x over previous
import functools

import numpy as np

import jax
import jax.numpy as jnp
from jax import lax
from jax.experimental import pallas as pl
from jax.experimental.pallas import tpu as pltpu
from jax.experimental.pallas import tpu_sc as plsc

N = 10000
E = 320000
D = 128
H = 128

NC = 2
NS = 16
NW = NC * NS

CHUNK = 128
CPT = 80
HALF = CPT // 2
EPT = CPT * CHUNK
E_PAD = EPT * NW

N_PAD = 10112

RPT = N_PAD // NS

_sc_mesh = plsc.VectorSubcoreMesh(core_axis_name="c", subcore_axis_name="s")


ZCOPY = 120


def _sc_agg_body(edge_hbm, feat_hbm, out_hbm,
                 src_v, dst_v, rows0, rows1, agg_sh, sem0, sem1):
    c = lax.axis_index("c")
    s = lax.axis_index("s")
    wid = c * NS + s

    zvec = jnp.zeros((16,), jnp.float32)

    SPL = 64

    def gather(j, buf, sem):
        pltpu.async_copy(feat_hbm.at[src_v.at[j, pl.ds(0, SPL)]],
                         buf.at[pl.ds(0, SPL)], sem)
        pltpu.async_copy(feat_hbm.at[src_v.at[j, pl.ds(SPL, CHUNK - SPL)]],
                         buf.at[pl.ds(SPL, CHUNK - SPL)], sem)

    def gather_wait(j, buf, sem):
        pltpu.make_async_copy(feat_hbm.at[src_v.at[j, pl.ds(0, SPL)]],
                              buf.at[pl.ds(0, SPL)], sem).wait()
        pltpu.make_async_copy(feat_hbm.at[src_v.at[j, pl.ds(SPL, CHUNK - SPL)]],
                              buf.at[pl.ds(SPL, CHUNK - SPL)], sem).wait()

    def scatter(j, buf):
        pltpu.sync_copy(buf, agg_sh.at[dst_v.at[j]], add=True)

    for h in range(CPT // HALF):
        pltpu.sync_copy(edge_hbm.at[0, wid, pl.ds(h * HALF, HALF)], src_v)
        pltpu.sync_copy(edge_hbm.at[1, wid, pl.ds(h * HALF, HALF)], dst_v)

        gather(0, rows0, sem0)

        if h == 0:
            def zrow(r, carry):
                for k in range(D // 16):
                    rows1[r, pl.ds(k * 16, 16)] = zvec
                return carry

            lax.fori_loop(0, ZCOPY, zrow, 0)
            base = s * RPT
            for z in range(RPT // ZCOPY):
                pltpu.sync_copy(rows1.at[pl.ds(0, ZCOPY)],
                                agg_sh.at[pl.ds(base + z * ZCOPY, ZCOPY)])
            rem = RPT % ZCOPY
            if rem:
                pltpu.sync_copy(rows1.at[pl.ds(0, rem)],
                                agg_sh.at[pl.ds(base + RPT - rem, rem)])
            plsc.subcore_barrier()

        gather(1, rows1, sem1)

        def step(i, carry):
            g = 2 * i
            gather_wait(g, rows0, sem0)
            scatter(g, rows0)
            gather(g + 2, rows0, sem0)
            gather_wait(g + 1, rows1, sem1)
            scatter(g + 1, rows1)
            gather(g + 3, rows1, sem1)
            return carry

        lax.fori_loop(0, HALF // 2 - 1, step, 0)

        gather_wait(HALF - 2, rows0, sem0)
        scatter(HALF - 2, rows0)
        gather_wait(HALF - 1, rows1, sem1)
        scatter(HALF - 1, rows1)

    plsc.subcore_barrier()
    pltpu.sync_copy(agg_sh.at[pl.ds(s * RPT, RPT)],
                    out_hbm.at[c, pl.ds(s * RPT, RPT)])


_sc_agg = pl.kernel(
    _sc_agg_body,
    out_type=jax.ShapeDtypeStruct((NC, N_PAD, D), jnp.float32),
    mesh=_sc_mesh,
    scratch_types=[
        pltpu.VMEM((HALF, CHUNK), jnp.int32),
        pltpu.VMEM((HALF, CHUNK), jnp.int32),
        pltpu.VMEM((CHUNK, D), jnp.float32),
        pltpu.VMEM((CHUNK, D), jnp.float32),
        pltpu.VMEM_SHARED((N_PAD, D), jnp.float32),
        pltpu.SemaphoreType.DMA,
        pltpu.SemaphoreType.DMA,
    ],
)


def _tc_mlp_body(feat_ref, agg_ref, w1_ref, b1_ref, w2_ref, b2_ref,
                 gamma_ref, beta_ref, out_ref):
    agg = agg_ref[0, :N, :] + agg_ref[1, :N, :]
    h = 2.0 * feat_ref[...] + agg
    h = jnp.dot(h, w1_ref[...], preferred_element_type=jnp.float32) + b1_ref[...]
    h = jnp.maximum(h, 0.0)
    h = jnp.dot(h, w2_ref[...], preferred_element_type=jnp.float32) + b2_ref[...]
    mean = jnp.mean(h, axis=0, keepdims=True)
    var = jnp.mean((h - mean) * (h - mean), axis=0, keepdims=True)
    h = (h - mean) * lax.rsqrt(var + 1e-5) * gamma_ref[...] + beta_ref[...]
    out_ref[...] = jnp.maximum(h, 0.0)


_tc_mlp = pl.pallas_call(
    _tc_mlp_body,
    out_shape=jax.ShapeDtypeStruct((N, H), jnp.float32),
)


_pad_n = E_PAD - E
_pad_i = np.arange(_pad_n, dtype=np.int32)
_PAD_EDGES = np.stack([_pad_i % N, N + _pad_i % (N_PAD - N)]).reshape(2, -1, CHUNK)


def kernel(features, edge_index, W1, b1, W2, b2, gamma, beta):
    edges = jnp.concatenate(
        [edge_index.reshape(2, E // CHUNK, CHUNK), jnp.asarray(_PAD_EDGES)],
        axis=1).reshape(2, NW, CPT, CHUNK)
    agg = _sc_agg(edges, features)
    return _tc_mlp(features, agg,
                   W1, b1.reshape(1, H), W2, b2.reshape(1, H),
                   gamma.reshape(1, H), beta.reshape(1, H))
